# baseline (speedup 1.0000x reference)
import functools

import jax
import jax.numpy as jnp
import numpy as np
from jax import lax
from jax.experimental import pallas as pl
from jax.experimental.pallas import tpu as pltpu
from jax.experimental.pallas import tpu_sc as plsc

_IDX = (2, 5, 7, 11, 13, 17, 19, 23, 29, 31, 37, 41, 43, 47, 53, 59, 61,
        67, 71, 73, 79, 83, 89, 91, 95, 97)

_B, _N, _D = 4096, 100, 64
_K = len(_IDX)
_NW = 32
_ROWS = _B * _K
_RPW = _ROWS // _NW
_CHUNK = 128
_NCHUNK = _RPW // _CHUNK

_FLAT_IDX = (
    (np.arange(_B, dtype=np.int32)[:, None] * _N
     + np.array(_IDX, dtype=np.int32)[None, :])
    .reshape(_NW, _NCHUNK, _CHUNK)
)


def _gather_body(x_hbm, idx_hbm, out_hbm, idx_v, rows_v, gsem, wsem):
    wid = lax.axis_index("s") * 2 + lax.axis_index("c")
    base = wid * _RPW
    pltpu.sync_copy(idx_hbm.at[wid], idx_v)

    def step(c, carry):
        pltpu.async_copy(x_hbm.at[idx_v.at[c]], rows_v, gsem).wait()
        pltpu.async_copy(rows_v,
                         out_hbm.at[pl.ds(base + c * _CHUNK, _CHUNK)],
                         wsem).wait()
        return carry

    lax.fori_loop(0, _NCHUNK, step, 0, unroll=False)


def kernel(inputs):
    x = inputs.reshape(_B * _N, _D)
    k = functools.partial(
        pl.kernel,
        mesh=plsc.VectorSubcoreMesh(core_axis_name="c", subcore_axis_name="s"),
        compiler_params=pltpu.CompilerParams(use_tc_tiling_on_sc=False),
        out_type=jax.ShapeDtypeStruct((_ROWS, _D), jnp.float32),
        scratch_types=[
            pltpu.VMEM((_NCHUNK, _CHUNK), jnp.int32),
            pltpu.VMEM((_CHUNK, _D), jnp.float32),
            pltpu.SemaphoreType.DMA,
            pltpu.SemaphoreType.DMA,
        ],
    )(_gather_body)
    out = k(x, jnp.asarray(_FLAT_IDX))
    return out.reshape(_B, _K, _D)


# speedup vs baseline: 1.0484x; 1.0484x over previous
import functools

import jax
import jax.numpy as jnp
import numpy as np
from jax import lax
from jax.experimental import pallas as pl
from jax.experimental.pallas import tpu as pltpu
from jax.experimental.pallas import tpu_sc as plsc

_IDX = (2, 5, 7, 11, 13, 17, 19, 23, 29, 31, 37, 41, 43, 47, 53, 59, 61,
        67, 71, 73, 79, 83, 89, 91, 95, 97)

_B, _N, _D = 4096, 100, 64
_K = len(_IDX)
_NW = 32
_ROWS = _B * _K
_RPW = _ROWS // _NW
_CHUNK = 128
_NCHUNK = _RPW // _CHUNK
_NBUF = 4

_FLAT_IDX = (
    (np.arange(_B, dtype=np.int32)[:, None] * _N
     + np.array(_IDX, dtype=np.int32)[None, :])
    .reshape(_NW, _NCHUNK, _CHUNK)
)


def _gather_body(x_hbm, idx_hbm, out_hbm, idx_v, rows_v,
                 g0, g1, g2, g3, w0, w1, w2, w3):
    gs = (g0, g1, g2, g3)
    ws = (w0, w1, w2, w3)
    wid = lax.axis_index("s") * 2 + lax.axis_index("c")
    base = wid * _RPW
    pltpu.sync_copy(idx_hbm.at[wid], idx_v)

    def wait_gather(k):
        pltpu.make_async_copy(x_hbm.at[pl.ds(0, _CHUNK)], rows_v.at[k],
                              gs[k]).wait()

    def wait_write(k):
        pltpu.make_async_copy(rows_v.at[k], out_hbm.at[pl.ds(0, _CHUNK)],
                              ws[k]).wait()

    def start_write(d, k):
        pltpu.async_copy(rows_v.at[k],
                         out_hbm.at[pl.ds(base + d * _CHUNK, _CHUNK)],
                         ws[k])

    def body(c, carry):
        for k in range(_NBUF):
            @pl.when(c % _NBUF == k)
            def _(k=k):
                @pl.when(c >= _NBUF)
                def _():
                    wait_write(k)
                pltpu.async_copy(x_hbm.at[idx_v.at[c]], rows_v.at[k], gs[k])
                j = (k + 1) % _NBUF
                @pl.when(c >= _NBUF - 1)
                def _():
                    wait_gather(j)
                    start_write(c - (_NBUF - 1), j)
        return carry

    lax.fori_loop(0, _NCHUNK, body, 0)
    for d in range(_NCHUNK - (_NBUF - 1), _NCHUNK):
        k = d % _NBUF
        wait_gather(k)
        start_write(d, k)
    for k in range(_NBUF):
        wait_write(k)


def kernel(inputs):
    x = inputs.reshape(_B * _N, _D)
    k = functools.partial(
        pl.kernel,
        mesh=plsc.VectorSubcoreMesh(core_axis_name="c", subcore_axis_name="s"),
        compiler_params=pltpu.CompilerParams(use_tc_tiling_on_sc=False),
        out_type=jax.ShapeDtypeStruct((_ROWS, _D), jnp.float32),
        scratch_types=[
            pltpu.VMEM((_NCHUNK, _CHUNK), jnp.int32),
            pltpu.VMEM((_NBUF, _CHUNK, _D), jnp.float32),
        ] + [pltpu.SemaphoreType.DMA] * (2 * _NBUF),
    )(_gather_body)
    out = k(x, jnp.asarray(_FLAT_IDX))
    return out.reshape(_B, _K, _D)
